# baseline (speedup 1.0000x reference)
import jax
import jax.numpy as jnp
from jax import lax
from jax.experimental import pallas as pl
from jax.experimental.pallas import tpu as pltpu
from jax.experimental.pallas import tpu_sc as plsc

N_GENES = 10000
WIDTH = 4
LATENT = 256
FAN_IN = 32
HIDDEN = N_GENES * WIDTH
NNZ1 = HIDDEN * FAN_IN
BATCH = 256
NEG_SLOPE = 0.01

CHUNK = 128
CN = CHUNK * FAN_IN
CW = LATENT * CHUNK
NVEC = CN // 16
N_CHUNKS = 320
NUM_WORKERS = 32
N_HALF = N_CHUNKS // 2
ROUNDS = N_HALF // NUM_WORKERS
TC_Q = 16
H_B = TC_Q * CHUNK
GT_B = H_B // WIDTH
STEPS_HALF = N_HALF // TC_Q
GENES_PAD = N_CHUNKS * CHUNK // WIDTH


def _make_sc_build(c0):
    def _sc_build(cm_hbm, wm_hbm, wt_hbm, cm_v, wm_v, buf0, buf1, ab0, ab1,
                  sem0, sem1, sem_c, sem_w):
        wid = lax.axis_index("s") * 2 + lax.axis_index("c")
        lane = lax.iota(jnp.int32, 16)
        bufs = (buf0, buf1)
        abs_ = (ab0, ab1)
        sems = (sem0, sem1)
        zero16 = jnp.zeros((16,), jnp.float32)

        for zb in bufs:
            def z0(i, carry, zb=zb):
                for j in range(16):
                    zb[pl.ds(i * 256 + j * 16, 16)] = zero16
                return carry

            lax.fori_loop(0, CW // 256, z0, 0)

        for t in range(ROUNDS):
            b = t % 2
            buf = bufs[b]
            ab = abs_[b]
            lcid = t * NUM_WORKERS + wid
            cid = c0 + lcid
            base = cid * CHUNK
            ngroups = jnp.clip((HIDDEN - base) // 16, 0, CHUNK // 16)
            stage = jnp.minimum(cid * CN, NNZ1 - CN)

            @pl.when(ngroups > 0)
            def _stage():
                pltpu.async_copy(cm_hbm.at[pl.ds(stage, CN)], cm_v, sem_c)
                pltpu.async_copy(wm_hbm.at[pl.ds(stage, CN)], wm_v, sem_w)

            if t >= 2:
                pltpu.make_async_copy(
                    buf, wt_hbm.at[pl.ds((lcid - 2 * NUM_WORKERS) * CW, CW)],
                    sems[b]).wait()

                def rzbody(v, carry):
                    for j in range(4):
                        addr = ab[pl.ds((v * 4 + j) * 16, 16)]
                        plsc.store_scatter(buf, [addr], zero16)
                    return carry

                lax.fori_loop(0, NVEC // 4, rzbody, 0)

            @pl.when(ngroups > 0)
            def _scatter():
                pltpu.make_async_copy(cm_hbm.at[pl.ds(stage, CN)], cm_v,
                                      sem_c).wait()
                pltpu.make_async_copy(wm_hbm.at[pl.ds(stage, CN)], wm_v,
                                      sem_w).wait()
                loc0 = base * FAN_IN - stage

                def gbody(g, carry):
                    node = g * 16 + lane
                    nnz = loc0 + node * FAN_IN
                    for k0 in range(0, FAN_IN, 4):
                        cs = [plsc.load_gather(cm_v, [nnz + (k0 + j)])
                              for j in range(4)]
                        ws = [plsc.load_gather(wm_v, [nnz + (k0 + j)])
                              for j in range(4)]
                        for j in range(4):
                            addr = cs[j] * CHUNK + node
                            plsc.addupdate_scatter(buf, [addr], ws[j])
                            ab[pl.ds((g * FAN_IN + k0 + j) * 16, 16)] = addr
                    return carry

                lax.fori_loop(0, ngroups, gbody, 0)

            pltpu.async_copy(buf, wt_hbm.at[pl.ds(lcid * CW, CW)], sems[b])

        pltpu.make_async_copy(
            bufs[(ROUNDS - 2) % 2],
            wt_hbm.at[pl.ds(((ROUNDS - 2) * NUM_WORKERS + wid) * CW, CW)],
            sems[(ROUNDS - 2) % 2]).wait()
        pltpu.make_async_copy(
            bufs[(ROUNDS - 1) % 2],
            wt_hbm.at[pl.ds(((ROUNDS - 1) * NUM_WORKERS + wid) * CW, CW)],
            sems[(ROUNDS - 1) % 2]).wait()

    return _sc_build


def _make_tc_body(step0, with_alias):
    def _tc_body(*refs):
        if with_alias:
            f_ref, wt_ref, b1_ref, w2_ref, b2_ref, _prev, out_ref, mask_ref = refs
        else:
            f_ref, wt_ref, b1_ref, w2_ref, b2_ref, out_ref, mask_ref = refs

        @pl.when(pl.program_id(0) == 0)
        def _fill_mask():
            hid_iota = lax.broadcasted_iota(jnp.int32, (H_B, GT_B), 0)
            gene_iota = lax.broadcasted_iota(jnp.int32, (H_B, GT_B), 1)
            mask_ref[...] = jnp.where(hid_iota // WIDTH == gene_iota, 1.0, 0.0)

        i = pl.program_id(0) + step0
        h = jnp.concatenate(
            [jnp.dot(f_ref[...], wt_ref[pl.ds(q * LATENT, LATENT), :],
                     preferred_element_type=jnp.float32)
             for q in range(TC_Q)], axis=1)
        h = h + b1_ref[...]
        h = jnp.where(h >= 0, h, NEG_SLOPE * h)
        h = h * w2_ref[...]
        nvalid = HIDDEN - i * H_B
        col = lax.broadcasted_iota(jnp.int32, (BATCH, H_B), 1)
        h = jnp.where(col < nvalid, h, 0.0)
        out_ref[...] = (jnp.dot(h, mask_ref[...],
                                preferred_element_type=jnp.float32)
                        + b2_ref[...])

    return _tc_body


def _sc_half(c0, conn1_col, w1):
    mesh = plsc.VectorSubcoreMesh(core_axis_name="c", subcore_axis_name="s")
    wt_flat = pl.kernel(
        _make_sc_build(c0),
        out_type=jax.ShapeDtypeStruct((N_HALF * CW,), jnp.float32),
        mesh=mesh,
        scratch_types=[
            pltpu.VMEM((CN,), jnp.int32),
            pltpu.VMEM((CN,), jnp.float32),
            pltpu.VMEM((CW,), jnp.float32),
            pltpu.VMEM((CW,), jnp.float32),
            pltpu.VMEM((CN,), jnp.int32),
            pltpu.VMEM((CN,), jnp.int32),
            pltpu.SemaphoreType.DMA,
            pltpu.SemaphoreType.DMA,
            pltpu.SemaphoreType.DMA,
            pltpu.SemaphoreType.DMA,
        ],
        compiler_params=pltpu.CompilerParams(needs_layout_passes=False),
    )(conn1_col, w1)
    return wt_flat.reshape(N_HALF * LATENT, CHUNK)


def _tc_half(step0, wt2, features, b1r, w2r, b2r, prev_out):
    with_alias = prev_out is not None
    in_specs = [
        pl.BlockSpec((BATCH, LATENT), lambda i: (0, 0)),
        pl.BlockSpec((TC_Q * LATENT, CHUNK), lambda i: (i, 0)),
        pl.BlockSpec((1, H_B), lambda i: (0, i + step0)),
        pl.BlockSpec((1, H_B), lambda i: (0, i + step0)),
        pl.BlockSpec((1, GT_B), lambda i: (0, i + step0)),
    ]
    args = [features, wt2, b1r, w2r, b2r]
    kwargs = {}
    if with_alias:
        in_specs.append(pl.BlockSpec(memory_space=pl.ANY))
        args.append(prev_out)
        kwargs["input_output_aliases"] = {5: 0}
    return pl.pallas_call(
        _make_tc_body(step0, with_alias),
        grid=(STEPS_HALF,),
        in_specs=in_specs,
        out_specs=pl.BlockSpec((BATCH, GT_B), lambda i, s0=step0: (0, i + s0)),
        out_shape=jax.ShapeDtypeStruct((BATCH, GENES_PAD), jnp.float32),
        scratch_shapes=[pltpu.VMEM((H_B, GT_B), jnp.float32)],
        **kwargs,
    )(*args)


def kernel(features, w1, b1, w2, b2, conn1_row, conn1_col, conn2_row, conn2_col):
    del conn1_row, conn2_row, conn2_col
    b1r = b1.reshape(1, HIDDEN)
    w2r = w2.reshape(1, HIDDEN)
    b2r = b2.reshape(1, N_GENES)

    wt2_a = _sc_half(0, conn1_col, w1)
    wt2_b = _sc_half(N_HALF, conn1_col, w1)
    out_a = _tc_half(0, wt2_a, features, b1r, w2r, b2r, None)
    out = _tc_half(STEPS_HALF, wt2_b, features, b1r, w2r, b2r, out_a)
    return out[:, :N_GENES]


# speedup vs baseline: 1.0723x; 1.0723x over previous
import jax
import jax.numpy as jnp
from jax import lax
from jax.experimental import pallas as pl
from jax.experimental.pallas import tpu as pltpu
from jax.experimental.pallas import tpu_sc as plsc

N_GENES = 10000
WIDTH = 4
LATENT = 256
FAN_IN = 32
HIDDEN = N_GENES * WIDTH
NNZ1 = HIDDEN * FAN_IN
BATCH = 256
NEG_SLOPE = 0.01

CHUNK = 128
CN = CHUNK * FAN_IN
CW = LATENT * CHUNK
NVEC = CN // 16
N_CHUNKS = 320
NUM_WORKERS = 32
N_HALF = N_CHUNKS // 2
ROUNDS = N_HALF // NUM_WORKERS
TC_Q = 16
H_B = TC_Q * CHUNK
GT_B = H_B // WIDTH
STEPS_HALF = N_HALF // TC_Q


def _make_sc_build(c0):
    def _sc_build(cm_hbm, wm_hbm, wt_hbm, cm_v, wm_v, buf0, buf1, ab0, ab1,
                  sem0, sem1, sem_c, sem_w):
        wid = lax.axis_index("s") * 2 + lax.axis_index("c")
        lane = lax.iota(jnp.int32, 16)
        bufs = (buf0, buf1)
        abs_ = (ab0, ab1)
        sems = (sem0, sem1)
        zero16 = jnp.zeros((16,), jnp.float32)

        for zb in bufs:
            def z0(i, carry, zb=zb):
                for j in range(16):
                    zb[pl.ds(i * 256 + j * 16, 16)] = zero16
                return carry

            lax.fori_loop(0, CW // 256, z0, 0)

        for t in range(ROUNDS):
            b = t % 2
            buf = bufs[b]
            ab = abs_[b]
            lcid = t * NUM_WORKERS + wid
            cid = c0 + lcid
            base = cid * CHUNK
            ngroups = jnp.clip((HIDDEN - base) // 16, 0, CHUNK // 16)
            stage = jnp.minimum(cid * CN, NNZ1 - CN)

            @pl.when(ngroups > 0)
            def _stage():
                pltpu.async_copy(cm_hbm.at[pl.ds(stage, CN)], cm_v, sem_c)
                pltpu.async_copy(wm_hbm.at[pl.ds(stage, CN)], wm_v, sem_w)

            if t >= 2:
                pltpu.make_async_copy(
                    buf, wt_hbm.at[pl.ds((lcid - 2 * NUM_WORKERS) * CW, CW)],
                    sems[b]).wait()

                def rzbody(v, carry):
                    for j in range(4):
                        addr = ab[pl.ds((v * 4 + j) * 16, 16)]
                        plsc.store_scatter(buf, [addr], zero16)
                    return carry

                lax.fori_loop(0, NVEC // 4, rzbody, 0)

            @pl.when(ngroups > 0)
            def _scatter():
                pltpu.make_async_copy(cm_hbm.at[pl.ds(stage, CN)], cm_v,
                                      sem_c).wait()
                pltpu.make_async_copy(wm_hbm.at[pl.ds(stage, CN)], wm_v,
                                      sem_w).wait()
                loc0 = base * FAN_IN - stage

                def gbody(g, carry):
                    node = g * 16 + lane
                    nnz = loc0 + node * FAN_IN
                    for k0 in range(0, FAN_IN, 4):
                        cs = [plsc.load_gather(cm_v, [nnz + (k0 + j)])
                              for j in range(4)]
                        ws = [plsc.load_gather(wm_v, [nnz + (k0 + j)])
                              for j in range(4)]
                        for j in range(4):
                            addr = cs[j] * CHUNK + node
                            plsc.addupdate_scatter(buf, [addr], ws[j])
                            ab[pl.ds((g * FAN_IN + k0 + j) * 16, 16)] = addr
                    return carry

                lax.fori_loop(0, ngroups, gbody, 0)

            pltpu.async_copy(buf, wt_hbm.at[pl.ds(lcid * CW, CW)], sems[b])

        pltpu.make_async_copy(
            bufs[(ROUNDS - 2) % 2],
            wt_hbm.at[pl.ds(((ROUNDS - 2) * NUM_WORKERS + wid) * CW, CW)],
            sems[(ROUNDS - 2) % 2]).wait()
        pltpu.make_async_copy(
            bufs[(ROUNDS - 1) % 2],
            wt_hbm.at[pl.ds(((ROUNDS - 1) * NUM_WORKERS + wid) * CW, CW)],
            sems[(ROUNDS - 1) % 2]).wait()

    return _sc_build


def _make_tc_body(step0, with_alias):
    def _tc_body(*refs):
        if with_alias:
            f_ref, wt_ref, b1_ref, w2_ref, b2_ref, _prev, out_ref, mask_ref = refs
        else:
            f_ref, wt_ref, b1_ref, w2_ref, b2_ref, out_ref, mask_ref = refs

        @pl.when(pl.program_id(0) == 0)
        def _fill_mask():
            hid_iota = lax.broadcasted_iota(jnp.int32, (H_B, GT_B), 0)
            gene_iota = lax.broadcasted_iota(jnp.int32, (H_B, GT_B), 1)
            mask_ref[...] = jnp.where(hid_iota // WIDTH == gene_iota, 1.0, 0.0)

        i = pl.program_id(0) + step0
        h = jnp.concatenate(
            [jnp.dot(f_ref[...], wt_ref[pl.ds(q * LATENT, LATENT), :],
                     preferred_element_type=jnp.float32)
             for q in range(TC_Q)], axis=1)
        h = h + b1_ref[...][None, :]
        h = jnp.where(h >= 0, h, NEG_SLOPE * h)
        h = h * w2_ref[...][None, :]
        nvalid = HIDDEN - i * H_B
        col = lax.broadcasted_iota(jnp.int32, (BATCH, H_B), 1)
        h = jnp.where(col < nvalid, h, 0.0)
        out_ref[...] = (jnp.dot(h, mask_ref[...],
                                preferred_element_type=jnp.float32)
                        + b2_ref[...][None, :])

    return _tc_body


def _sc_half(c0, conn1_col, w1):
    mesh = plsc.VectorSubcoreMesh(core_axis_name="c", subcore_axis_name="s")
    wt_flat = pl.kernel(
        _make_sc_build(c0),
        out_type=jax.ShapeDtypeStruct((N_HALF * CW,), jnp.float32),
        mesh=mesh,
        scratch_types=[
            pltpu.VMEM((CN,), jnp.int32),
            pltpu.VMEM((CN,), jnp.float32),
            pltpu.VMEM((CW,), jnp.float32),
            pltpu.VMEM((CW,), jnp.float32),
            pltpu.VMEM((CN,), jnp.int32),
            pltpu.VMEM((CN,), jnp.int32),
            pltpu.SemaphoreType.DMA,
            pltpu.SemaphoreType.DMA,
            pltpu.SemaphoreType.DMA,
            pltpu.SemaphoreType.DMA,
        ],
        compiler_params=pltpu.CompilerParams(needs_layout_passes=False),
    )(conn1_col, w1)
    return wt_flat.reshape(N_HALF * LATENT, CHUNK)


def _tc_half(step0, wt2, features, b1r, w2r, b2r, prev_out):
    with_alias = prev_out is not None
    in_specs = [
        pl.BlockSpec((BATCH, LATENT), lambda i: (0, 0)),
        pl.BlockSpec((TC_Q * LATENT, CHUNK), lambda i: (i, 0)),
        pl.BlockSpec((H_B,), lambda i: (i + step0,)),
        pl.BlockSpec((H_B,), lambda i: (i + step0,)),
        pl.BlockSpec((GT_B,), lambda i: (i + step0,)),
    ]
    args = [features, wt2, b1r, w2r, b2r]
    kwargs = {}
    if with_alias:
        in_specs.append(pl.BlockSpec(memory_space=pl.ANY))
        args.append(prev_out)
        kwargs["input_output_aliases"] = {5: 0}
    return pl.pallas_call(
        _make_tc_body(step0, with_alias),
        grid=(STEPS_HALF,),
        in_specs=in_specs,
        out_specs=pl.BlockSpec((BATCH, GT_B), lambda i, s0=step0: (0, i + s0)),
        out_shape=jax.ShapeDtypeStruct((BATCH, N_GENES), jnp.float32),
        scratch_shapes=[pltpu.VMEM((H_B, GT_B), jnp.float32)],
        **kwargs,
    )(*args)


def kernel(features, w1, b1, w2, b2, conn1_row, conn1_col, conn2_row, conn2_col):
    del conn1_row, conn2_row, conn2_col
    b1r, w2r, b2r = b1, w2, b2

    wt2_a = _sc_half(0, conn1_col, w1)
    wt2_b = _sc_half(N_HALF, conn1_col, w1)
    out_a = _tc_half(0, wt2_a, features, b1r, w2r, b2r, None)
    out = _tc_half(STEPS_HALF, wt2_b, features, b1r, w2r, b2r, out_a)
    return out


# speedup vs baseline: 1.0914x; 1.0178x over previous
import jax
import jax.numpy as jnp
from jax import lax
from jax.experimental import pallas as pl
from jax.experimental.pallas import tpu as pltpu
from jax.experimental.pallas import tpu_sc as plsc

N_GENES = 10000
WIDTH = 4
LATENT = 256
FAN_IN = 32
HIDDEN = N_GENES * WIDTH
NNZ1 = HIDDEN * FAN_IN
BATCH = 256
NEG_SLOPE = 0.01

CHUNK = 128
CN = CHUNK * FAN_IN
CW = LATENT * CHUNK
NVEC = CN // 16
N_CHUNKS = 320
NUM_WORKERS = 32
N_A = 192
N_B = N_CHUNKS - N_A
TC_Q = 16
H_B = TC_Q * CHUNK
GT_B = H_B // WIDTH


def _make_sc_build(c0, nchunks):
    rounds = nchunks // NUM_WORKERS

    def _sc_build(cm_hbm, wm_hbm, wt_hbm, cm_v, wm_v, buf0, buf1, ab0, ab1,
                  sem0, sem1, sem_c, sem_w):
        wid = lax.axis_index("s") * 2 + lax.axis_index("c")
        lane = lax.iota(jnp.int32, 16)
        bufs = (buf0, buf1)
        abs_ = (ab0, ab1)
        sems = (sem0, sem1)
        zero16 = jnp.zeros((16,), jnp.float32)

        for zb in bufs:
            def z0(i, carry, zb=zb):
                for j in range(16):
                    zb[pl.ds(i * 256 + j * 16, 16)] = zero16
                return carry

            lax.fori_loop(0, CW // 256, z0, 0)

        for t in range(rounds):
            b = t % 2
            buf = bufs[b]
            ab = abs_[b]
            lcid = t * NUM_WORKERS + wid
            cid = c0 + lcid
            base = cid * CHUNK
            ngroups = jnp.clip((HIDDEN - base) // 16, 0, CHUNK // 16)
            stage = jnp.minimum(cid * CN, NNZ1 - CN)

            @pl.when(ngroups > 0)
            def _stage():
                pltpu.async_copy(cm_hbm.at[pl.ds(stage, CN)], cm_v, sem_c)
                pltpu.async_copy(wm_hbm.at[pl.ds(stage, CN)], wm_v, sem_w)

            if t >= 2:
                pltpu.make_async_copy(
                    buf, wt_hbm.at[pl.ds((lcid - 2 * NUM_WORKERS) * CW, CW)],
                    sems[b]).wait()

                def rzbody(v, carry):
                    for j in range(4):
                        addr = ab[pl.ds((v * 4 + j) * 16, 16)]
                        plsc.store_scatter(buf, [addr], zero16)
                    return carry

                lax.fori_loop(0, NVEC // 4, rzbody, 0)

            @pl.when(ngroups > 0)
            def _scatter():
                pltpu.make_async_copy(cm_hbm.at[pl.ds(stage, CN)], cm_v,
                                      sem_c).wait()
                pltpu.make_async_copy(wm_hbm.at[pl.ds(stage, CN)], wm_v,
                                      sem_w).wait()
                loc0 = base * FAN_IN - stage

                def gbody(g, carry):
                    node = g * 16 + lane
                    nnz = loc0 + node * FAN_IN
                    for k0 in range(0, FAN_IN, 4):
                        cs = [plsc.load_gather(cm_v, [nnz + (k0 + j)])
                              for j in range(4)]
                        ws = [plsc.load_gather(wm_v, [nnz + (k0 + j)])
                              for j in range(4)]
                        for j in range(4):
                            addr = cs[j] * CHUNK + node
                            plsc.addupdate_scatter(buf, [addr], ws[j])
                            ab[pl.ds((g * FAN_IN + k0 + j) * 16, 16)] = addr
                    return carry

                lax.fori_loop(0, ngroups, gbody, 0)

            pltpu.async_copy(buf, wt_hbm.at[pl.ds(lcid * CW, CW)], sems[b])

        pltpu.make_async_copy(
            bufs[(rounds - 2) % 2],
            wt_hbm.at[pl.ds(((rounds - 2) * NUM_WORKERS + wid) * CW, CW)],
            sems[(rounds - 2) % 2]).wait()
        pltpu.make_async_copy(
            bufs[(rounds - 1) % 2],
            wt_hbm.at[pl.ds(((rounds - 1) * NUM_WORKERS + wid) * CW, CW)],
            sems[(rounds - 1) % 2]).wait()

    return _sc_build


def _make_tc_body(step0, with_alias):
    def _tc_body(*refs):
        if with_alias:
            f_ref, wt_ref, b1_ref, w2_ref, b2_ref, _prev, out_ref, mask_ref = refs
        else:
            f_ref, wt_ref, b1_ref, w2_ref, b2_ref, out_ref, mask_ref = refs

        @pl.when(pl.program_id(0) == 0)
        def _fill_mask():
            hid_iota = lax.broadcasted_iota(jnp.int32, (H_B, GT_B), 0)
            gene_iota = lax.broadcasted_iota(jnp.int32, (H_B, GT_B), 1)
            mask_ref[...] = jnp.where(hid_iota // WIDTH == gene_iota, 1.0, 0.0)

        i = pl.program_id(0) + step0
        h = jnp.concatenate(
            [jnp.dot(f_ref[...], wt_ref[pl.ds(q * LATENT, LATENT), :],
                     preferred_element_type=jnp.float32)
             for q in range(TC_Q)], axis=1)
        h = h + b1_ref[...][None, :]
        h = jnp.where(h >= 0, h, NEG_SLOPE * h)
        h = h * w2_ref[...][None, :]
        nvalid = HIDDEN - i * H_B
        col = lax.broadcasted_iota(jnp.int32, (BATCH, H_B), 1)
        h = jnp.where(col < nvalid, h, 0.0)
        out_ref[...] = (jnp.dot(h, mask_ref[...],
                                preferred_element_type=jnp.float32)
                        + b2_ref[...][None, :])

    return _tc_body


def _sc_half(c0, nchunks, conn1_col, w1):
    mesh = plsc.VectorSubcoreMesh(core_axis_name="c", subcore_axis_name="s")
    wt_flat = pl.kernel(
        _make_sc_build(c0, nchunks),
        out_type=jax.ShapeDtypeStruct((nchunks * CW,), jnp.float32),
        mesh=mesh,
        scratch_types=[
            pltpu.VMEM((CN,), jnp.int32),
            pltpu.VMEM((CN,), jnp.float32),
            pltpu.VMEM((CW,), jnp.float32),
            pltpu.VMEM((CW,), jnp.float32),
            pltpu.VMEM((CN,), jnp.int32),
            pltpu.VMEM((CN,), jnp.int32),
            pltpu.SemaphoreType.DMA,
            pltpu.SemaphoreType.DMA,
            pltpu.SemaphoreType.DMA,
            pltpu.SemaphoreType.DMA,
        ],
        compiler_params=pltpu.CompilerParams(needs_layout_passes=False),
    )(conn1_col, w1)
    return wt_flat.reshape(nchunks * LATENT, CHUNK)


def _tc_half(step0, nsteps, wt2, features, b1r, w2r, b2r, prev_out):
    with_alias = prev_out is not None
    in_specs = [
        pl.BlockSpec((BATCH, LATENT), lambda i: (0, 0)),
        pl.BlockSpec((TC_Q * LATENT, CHUNK), lambda i: (i, 0)),
        pl.BlockSpec((H_B,), lambda i: (i + step0,)),
        pl.BlockSpec((H_B,), lambda i: (i + step0,)),
        pl.BlockSpec((GT_B,), lambda i: (i + step0,)),
    ]
    args = [features, wt2, b1r, w2r, b2r]
    kwargs = {}
    if with_alias:
        in_specs.append(pl.BlockSpec(memory_space=pl.ANY))
        args.append(prev_out)
        kwargs["input_output_aliases"] = {5: 0}
    return pl.pallas_call(
        _make_tc_body(step0, with_alias),
        grid=(nsteps,),
        in_specs=in_specs,
        out_specs=pl.BlockSpec((BATCH, GT_B), lambda i, s0=step0: (0, i + s0)),
        out_shape=jax.ShapeDtypeStruct((BATCH, N_GENES), jnp.float32),
        scratch_shapes=[pltpu.VMEM((H_B, GT_B), jnp.float32)],
        **kwargs,
    )(*args)


def kernel(features, w1, b1, w2, b2, conn1_row, conn1_col, conn2_row, conn2_col):
    del conn1_row, conn2_row, conn2_col
    b1r, w2r, b2r = b1, w2, b2

    wt2_a = _sc_half(0, N_A, conn1_col, w1)
    wt2_b = _sc_half(N_A, N_B, conn1_col, w1)
    out_a = _tc_half(0, N_A // TC_Q, wt2_a, features, b1r, w2r, b2r, None)
    out = _tc_half(N_A // TC_Q, N_B // TC_Q, wt2_b, features, b1r, w2r, b2r,
                   out_a)
    return out


# speedup vs baseline: 1.1001x; 1.0079x over previous
import jax
import jax.numpy as jnp
from jax import lax
from jax.experimental import pallas as pl
from jax.experimental.pallas import tpu as pltpu
from jax.experimental.pallas import tpu_sc as plsc

N_GENES = 10000
WIDTH = 4
LATENT = 256
FAN_IN = 32
HIDDEN = N_GENES * WIDTH
NNZ1 = HIDDEN * FAN_IN
BATCH = 256
NEG_SLOPE = 0.01

CHUNK = 128
CN = CHUNK * FAN_IN
CW = LATENT * CHUNK
NVEC = CN // 16
N_CHUNKS = 320
NUM_WORKERS = 32
N_A = 224
N_B = N_CHUNKS - N_A
TC_Q = 16
H_B = TC_Q * CHUNK
GT_B = H_B // WIDTH


def _make_sc_build(c0, nchunks):
    rounds = nchunks // NUM_WORKERS

    def _sc_build(cm_hbm, wm_hbm, wt_hbm, cm_v, wm_v, buf0, buf1, ab0, ab1,
                  sem0, sem1, sem_c, sem_w):
        wid = lax.axis_index("s") * 2 + lax.axis_index("c")
        lane = lax.iota(jnp.int32, 16)
        bufs = (buf0, buf1)
        abs_ = (ab0, ab1)
        sems = (sem0, sem1)
        zero16 = jnp.zeros((16,), jnp.float32)

        for zb in bufs:
            def z0(i, carry, zb=zb):
                for j in range(16):
                    zb[pl.ds(i * 256 + j * 16, 16)] = zero16
                return carry

            lax.fori_loop(0, CW // 256, z0, 0)

        for t in range(rounds):
            b = t % 2
            buf = bufs[b]
            ab = abs_[b]
            lcid = t * NUM_WORKERS + wid
            cid = c0 + lcid
            base = cid * CHUNK
            ngroups = jnp.clip((HIDDEN - base) // 16, 0, CHUNK // 16)
            stage = jnp.minimum(cid * CN, NNZ1 - CN)

            @pl.when(ngroups > 0)
            def _stage():
                pltpu.async_copy(cm_hbm.at[pl.ds(stage, CN)], cm_v, sem_c)
                pltpu.async_copy(wm_hbm.at[pl.ds(stage, CN)], wm_v, sem_w)

            if t >= 2:
                pltpu.make_async_copy(
                    buf, wt_hbm.at[pl.ds((lcid - 2 * NUM_WORKERS) * CW, CW)],
                    sems[b]).wait()

                def rzbody(v, carry):
                    for j in range(4):
                        addr = ab[pl.ds((v * 4 + j) * 16, 16)]
                        plsc.store_scatter(buf, [addr], zero16)
                    return carry

                lax.fori_loop(0, NVEC // 4, rzbody, 0)

            @pl.when(ngroups > 0)
            def _scatter():
                pltpu.make_async_copy(cm_hbm.at[pl.ds(stage, CN)], cm_v,
                                      sem_c).wait()
                pltpu.make_async_copy(wm_hbm.at[pl.ds(stage, CN)], wm_v,
                                      sem_w).wait()
                loc0 = base * FAN_IN - stage

                def gbody(g, carry):
                    node = g * 16 + lane
                    nnz = loc0 + node * FAN_IN
                    for k0 in range(0, FAN_IN, 4):
                        cs = [plsc.load_gather(cm_v, [nnz + (k0 + j)])
                              for j in range(4)]
                        ws = [plsc.load_gather(wm_v, [nnz + (k0 + j)])
                              for j in range(4)]
                        for j in range(4):
                            addr = cs[j] * CHUNK + node
                            plsc.addupdate_scatter(buf, [addr], ws[j])
                            ab[pl.ds((g * FAN_IN + k0 + j) * 16, 16)] = addr
                    return carry

                lax.fori_loop(0, ngroups, gbody, 0)

            pltpu.async_copy(buf, wt_hbm.at[pl.ds(lcid * CW, CW)], sems[b])

        pltpu.make_async_copy(
            bufs[(rounds - 2) % 2],
            wt_hbm.at[pl.ds(((rounds - 2) * NUM_WORKERS + wid) * CW, CW)],
            sems[(rounds - 2) % 2]).wait()
        pltpu.make_async_copy(
            bufs[(rounds - 1) % 2],
            wt_hbm.at[pl.ds(((rounds - 1) * NUM_WORKERS + wid) * CW, CW)],
            sems[(rounds - 1) % 2]).wait()

    return _sc_build


def _make_tc_body(step0, with_alias):
    def _tc_body(*refs):
        if with_alias:
            f_ref, wt_ref, b1_ref, w2_ref, b2_ref, _prev, out_ref, mask_ref = refs
        else:
            f_ref, wt_ref, b1_ref, w2_ref, b2_ref, out_ref, mask_ref = refs

        @pl.when(pl.program_id(0) == 0)
        def _fill_mask():
            hid_iota = lax.broadcasted_iota(jnp.int32, (H_B, GT_B), 0)
            gene_iota = lax.broadcasted_iota(jnp.int32, (H_B, GT_B), 1)
            mask_ref[...] = jnp.where(hid_iota // WIDTH == gene_iota, 1.0, 0.0)

        i = pl.program_id(0) + step0
        h = jnp.concatenate(
            [jnp.dot(f_ref[...], wt_ref[pl.ds(q * LATENT, LATENT), :],
                     preferred_element_type=jnp.float32)
             for q in range(TC_Q)], axis=1)
        h = h + b1_ref[...][None, :]
        h = jnp.where(h >= 0, h, NEG_SLOPE * h)
        h = h * w2_ref[...][None, :]
        nvalid = HIDDEN - i * H_B
        col = lax.broadcasted_iota(jnp.int32, (BATCH, H_B), 1)
        h = jnp.where(col < nvalid, h, 0.0)
        out_ref[...] = (jnp.dot(h, mask_ref[...],
                                preferred_element_type=jnp.float32)
                        + b2_ref[...][None, :])

    return _tc_body


def _sc_half(c0, nchunks, conn1_col, w1):
    mesh = plsc.VectorSubcoreMesh(core_axis_name="c", subcore_axis_name="s")
    wt_flat = pl.kernel(
        _make_sc_build(c0, nchunks),
        out_type=jax.ShapeDtypeStruct((nchunks * CW,), jnp.float32),
        mesh=mesh,
        scratch_types=[
            pltpu.VMEM((CN,), jnp.int32),
            pltpu.VMEM((CN,), jnp.float32),
            pltpu.VMEM((CW,), jnp.float32),
            pltpu.VMEM((CW,), jnp.float32),
            pltpu.VMEM((CN,), jnp.int32),
            pltpu.VMEM((CN,), jnp.int32),
            pltpu.SemaphoreType.DMA,
            pltpu.SemaphoreType.DMA,
            pltpu.SemaphoreType.DMA,
            pltpu.SemaphoreType.DMA,
        ],
        compiler_params=pltpu.CompilerParams(needs_layout_passes=False),
    )(conn1_col, w1)
    return wt_flat.reshape(nchunks * LATENT, CHUNK)


def _tc_half(step0, nsteps, wt2, features, b1r, w2r, b2r, prev_out):
    with_alias = prev_out is not None
    in_specs = [
        pl.BlockSpec((BATCH, LATENT), lambda i: (0, 0)),
        pl.BlockSpec((TC_Q * LATENT, CHUNK), lambda i: (i, 0)),
        pl.BlockSpec((H_B,), lambda i: (i + step0,)),
        pl.BlockSpec((H_B,), lambda i: (i + step0,)),
        pl.BlockSpec((GT_B,), lambda i: (i + step0,)),
    ]
    args = [features, wt2, b1r, w2r, b2r]
    kwargs = {}
    if with_alias:
        in_specs.append(pl.BlockSpec(memory_space=pl.ANY))
        args.append(prev_out)
        kwargs["input_output_aliases"] = {5: 0}
    return pl.pallas_call(
        _make_tc_body(step0, with_alias),
        grid=(nsteps,),
        in_specs=in_specs,
        out_specs=pl.BlockSpec((BATCH, GT_B), lambda i, s0=step0: (0, i + s0)),
        out_shape=jax.ShapeDtypeStruct((BATCH, N_GENES), jnp.float32),
        scratch_shapes=[pltpu.VMEM((H_B, GT_B), jnp.float32)],
        **kwargs,
    )(*args)


def kernel(features, w1, b1, w2, b2, conn1_row, conn1_col, conn2_row, conn2_col):
    del conn1_row, conn2_row, conn2_col
    b1r, w2r, b2r = b1, w2, b2

    wt2_a = _sc_half(0, N_A, conn1_col, w1)
    wt2_b = _sc_half(N_A, N_B, conn1_col, w1)
    out_a = _tc_half(0, N_A // TC_Q, wt2_a, features, b1r, w2r, b2r, None)
    out = _tc_half(N_A // TC_Q, N_B // TC_Q, wt2_b, features, b1r, w2r, b2r,
                   out_a)
    return out
